# baseline (speedup 1.0000x reference)
import functools

import jax
import jax.numpy as jnp
from jax import lax
from jax.experimental import pallas as pl
from jax.experimental.pallas import tpu as pltpu
from jax.experimental.pallas import tpu_sc as plsc

E_DIM = 64
PAIR = 2 * E_DIM
NC = 2
NS = 16
L = 16
NW = NC * NS
BLK = 128
NGRP = BLK // L


def _rsqrt(x):
    i = plsc.bitcast(x, jnp.int32)
    i = jnp.int32(0x5F3759DF) - lax.shift_right_logical(i, 1)
    y = plsc.bitcast(i, jnp.float32)
    half_x = x * jnp.float32(0.5)
    for _ in range(3):
        y = y * (jnp.float32(1.5) - half_x * y * y)
    return y


def _make_sc_call(batch):
    per_w = batch // NW
    nblk = per_w // BLK
    mesh = plsc.VectorSubcoreMesh(
        core_axis_name="c", subcore_axis_name="s", num_cores=NC, num_subcores=NS
    )

    @functools.partial(
        pl.kernel,
        out_type=jax.ShapeDtypeStruct((batch,), jnp.float32),
        mesh=mesh,
        compiler_params=pltpu.CompilerParams(
            needs_layout_passes=False, use_tc_tiling_on_sc=True
        ),
        scratch_types=[
            pltpu.VMEM((3, nblk, BLK), jnp.int32),
            pltpu.VMEM((3, nblk, BLK), jnp.int32),
            pltpu.VMEM((2, 3, BLK, PAIR), jnp.float32),
            pltpu.VMEM((per_w,), jnp.float32),
            pltpu.SemaphoreType.DMA,
            pltpu.SemaphoreType.DMA,
        ],
    )
    def sc_call(
        idx_hbm, ent_hbm, rel_hbm, out_hbm, idx_v, idx2_v, rows, out_v, sem0, sem1
    ):
        sems = (sem0, sem1)
        wid = lax.axis_index("s") * NC + lax.axis_index("c")
        base = wid * per_w

        for r in range(3):
            pltpu.sync_copy(idx_hbm.at[r, pl.ds(wid * nblk, nblk)], idx_v.at[r])

        for r in range(3):
            for j in range(nblk):
                for i in range(BLK // L):
                    v = idx_v.at[r, j][pl.ds(i * L, L)]
                    idx2_v.at[r, j][pl.ds(i * L, L)] = lax.shift_right_logical(v, 1)

        def issue(j, slot):
            cps = []
            for r, tab in ((0, ent_hbm), (1, rel_hbm), (2, ent_hbm)):
                cps.append(
                    pltpu.async_copy(
                        tab.at[idx2_v.at[r, j]],
                        rows.at[slot, r],
                        sems[slot],
                    )
                )
            return cps

        lane = lax.iota(jnp.int32, L)

        def compute(j, slot):
            sbuf = rows.at[slot, 0]
            pbuf = rows.at[slot, 1]
            obuf = rows.at[slot, 2]

            def group(g, _):
                rowid = lane + g * jnp.int32(L)
                cols = []
                for r in range(3):
                    iv = idx_v.at[r, j][pl.ds(g * L, L)]
                    cols.append(
                        lax.shift_left(iv & jnp.int32(1), jnp.int32(6))
                    )
                scol, pcol, ocol = cols
                z = jnp.zeros((L,), jnp.float32)
                dot, ns, np_, no = z, z, z, z
                for k in range(E_DIM):
                    kv = jnp.int32(k)
                    sv = plsc.load_gather(sbuf, [rowid, scol + kv])
                    pv = plsc.load_gather(pbuf, [rowid, pcol + kv])
                    ov = plsc.load_gather(obuf, [rowid, ocol + kv])
                    sp = sv * pv
                    dot = dot + sp * ov
                    ns = ns + sv * sv
                    np_ = np_ + pv * pv
                    no = no + ov * ov
                eps = jnp.float32(1e-12)
                prod = (
                    jnp.maximum(ns, eps)
                    * jnp.maximum(np_, eps)
                    * jnp.maximum(no, eps)
                )
                out_v[pl.ds(j * BLK + g * L, L)] = dot * _rsqrt(prod)
                return 0

            lax.fori_loop(0, NGRP, group, 0)

        inflight = issue(0, 0)
        for j in range(nblk):
            slot = j % 2
            for c in inflight:
                c.wait()
            if j + 1 < nblk:
                inflight = issue(j + 1, (j + 1) % 2)
            compute(j, slot)

        pltpu.sync_copy(out_v, out_hbm.at[pl.ds(base, per_w)])

    return sc_call


@jax.jit
def kernel(inputs, entity_table, rel_table):
    batch = inputs.shape[0]
    per_w = batch // NW
    nblk = per_w // BLK
    idx = jnp.transpose(inputs).reshape(3, NW * nblk, BLK)
    ent2 = jnp.reshape(entity_table, (entity_table.shape[0] // 2, PAIR))
    rel2 = jnp.reshape(rel_table, (rel_table.shape[0] // 2, PAIR))
    scores = _make_sc_call(batch)(idx, ent2, rel2)
    return scores.reshape(batch, 1)


# speedup vs baseline: 1.7827x; 1.7827x over previous
import functools

import jax
import jax.numpy as jnp
from jax import lax
from jax.experimental import pallas as pl
from jax.experimental.pallas import tpu as pltpu
from jax.experimental.pallas import tpu_sc as plsc

E_DIM = 64
NC = 2
NS = 16
L = 16
NW = NC * NS
CH = 2048
UNITS = 4


def _make_gather_call(n_rows, batch):
    nch = batch // CH
    qp = CH // L
    mesh = plsc.VectorSubcoreMesh(
        core_axis_name="c", subcore_axis_name="s", num_cores=NC, num_subcores=NS
    )

    @functools.partial(
        pl.kernel,
        out_type=jax.ShapeDtypeStruct((3, E_DIM, batch), jnp.float32),
        mesh=mesh,
        compiler_params=pltpu.CompilerParams(
            needs_layout_passes=False, use_tc_tiling_on_sc=True
        ),
        scratch_types=[
            pltpu.VMEM((1, n_rows), jnp.float32),
            pltpu.VMEM((2, 2, 1, CH), jnp.int32),
            pltpu.VMEM((2, 2, 1, CH), jnp.float32),
            pltpu.SemaphoreType.DMA,
            pltpu.SemaphoreType.DMA,
            pltpu.SemaphoreType.DMA,
        ],
    )
    def gcall(ent_t, rel_t, idx_t, v_hbm, col_v, idx_c, out_c, semc, semi, semo):
        wid = lax.axis_index("s") * NC + lax.axis_index("c")

        def run_unit(tab, col, roles):
            pltpu.async_copy(tab.at[pl.ds(col, 1), :], col_v, semc).wait()

            def fire_idx(j, slot):
                for r_i, role in enumerate(roles):
                    pltpu.async_copy(
                        idx_t.at[role, pl.ds(0, 1), pl.ds(j * CH, CH)],
                        idx_c.at[r_i, slot],
                        semi,
                    )

            fire_idx(0, 0)

            def chunk_body(j, _):
                slot = lax.rem(j, 2)

                @pl.when(j + 1 < nch)
                def _():
                    fire_idx(j + 1, 1 - slot)

                for r_i, role in enumerate(roles):
                    pltpu.make_async_copy(
                        idx_t.at[role, pl.ds(0, 1), pl.ds(j * CH, CH)],
                        idx_c.at[r_i, slot],
                        semi,
                    ).wait()

                for r_i, role in enumerate(roles):
                    @pl.when(j >= 2)
                    def _():
                        pltpu.make_async_copy(
                            out_c.at[r_i, slot],
                            v_hbm.at[role, pl.ds(col, 1), pl.ds((j - 2) * CH, CH)],
                            semo,
                        ).wait()

                    for q in range(qp):
                        iv = idx_c[r_i, slot, 0, pl.ds(q * L, L)]
                        out_c[r_i, slot, 0, pl.ds(q * L, L)] = plsc.load_gather(
                            col_v.at[0], [iv]
                        )
                    pltpu.async_copy(
                        out_c.at[r_i, slot],
                        v_hbm.at[role, pl.ds(col, 1), pl.ds(j * CH, CH)],
                        semo,
                    )
                return 0

            lax.fori_loop(0, nch, chunk_body, 0)

            for j in (nch - 2, nch - 1):
                slot = j % 2
                for r_i, role in enumerate(roles):
                    pltpu.make_async_copy(
                        out_c.at[r_i, slot],
                        v_hbm.at[role, pl.ds(col, 1), pl.ds(j * CH, CH)],
                        semo,
                    ).wait()

        def unit_body(i, _):
            u = wid * UNITS + i
            is_ent = u < E_DIM

            @pl.when(is_ent)
            def _():
                run_unit(ent_t, u, (0, 2))

            @pl.when(jnp.logical_not(is_ent))
            def _():
                run_unit(rel_t, u - E_DIM, (1,))

            return 0

        lax.fori_loop(0, UNITS, unit_body, 0)

    return gcall


def _make_score_call(batch):
    blk = 512

    def body(vref, oref):
        s = vref[0]
        p = vref[1]
        o = vref[2]
        eps = jnp.float32(1e-12)
        dot = jnp.sum(s * p * o, axis=0, keepdims=True)
        ns = jnp.maximum(jnp.sum(s * s, axis=0, keepdims=True), eps)
        np_ = jnp.maximum(jnp.sum(p * p, axis=0, keepdims=True), eps)
        no = jnp.maximum(jnp.sum(o * o, axis=0, keepdims=True), eps)
        oref[...] = dot * lax.rsqrt(ns) * lax.rsqrt(np_) * lax.rsqrt(no)

    return pl.pallas_call(
        body,
        grid=(batch // blk,),
        in_specs=[pl.BlockSpec((3, E_DIM, blk), lambda i: (0, 0, i))],
        out_specs=pl.BlockSpec((1, blk), lambda i: (0, i)),
        out_shape=jax.ShapeDtypeStruct((1, batch), jnp.float32),
    )


@jax.jit
def kernel(inputs, entity_table, rel_table):
    batch = inputs.shape[0]
    idx_t = jnp.transpose(inputs).reshape(3, 1, batch)
    ent_t = jnp.transpose(entity_table)
    rel_t = jnp.transpose(rel_table)
    v = _make_gather_call(entity_table.shape[0], batch)(ent_t, rel_t, idx_t)
    scores = _make_score_call(batch)(v)
    return jnp.transpose(scores)


# speedup vs baseline: 1.9852x; 1.1136x over previous
import functools

import jax
import jax.numpy as jnp
from jax import lax
from jax.experimental import pallas as pl
from jax.experimental.pallas import tpu as pltpu
from jax.experimental.pallas import tpu_sc as plsc

E_DIM = 64
NC = 2
NS = 16
L = 16
NW = NC * NS
CH = 2048
UNITS = 4


def _make_gather_call(n_rows, batch):
    nch = batch // CH
    qp = CH // L
    mesh = plsc.VectorSubcoreMesh(
        core_axis_name="c", subcore_axis_name="s", num_cores=NC, num_subcores=NS
    )

    @functools.partial(
        pl.kernel,
        out_type=jax.ShapeDtypeStruct((3, E_DIM, batch), jnp.float32),
        mesh=mesh,
        compiler_params=pltpu.CompilerParams(
            needs_layout_passes=False, use_tc_tiling_on_sc=True
        ),
        scratch_types=[
            pltpu.VMEM((1, n_rows), jnp.float32),
            pltpu.VMEM((2, 2, 1, CH), jnp.int32),
            pltpu.VMEM((2, 2, 1, CH), jnp.float32),
            pltpu.SemaphoreType.DMA,
            pltpu.SemaphoreType.DMA,
            pltpu.SemaphoreType.DMA,
        ],
    )
    def gcall(ent_t, rel_t, idx_t, v_hbm, col_v, idx_c, out_c, semc, semi, semo):
        wid = lax.axis_index("s") * NC + lax.axis_index("c")

        def run_unit(tab, col, roles):
            pltpu.async_copy(tab.at[pl.ds(col, 1), :], col_v, semc).wait()

            def fire_idx(j, slot):
                for r_i, role in enumerate(roles):
                    pltpu.async_copy(
                        idx_t.at[role, pl.ds(0, 1), pl.ds(j * CH, CH)],
                        idx_c.at[r_i, slot],
                        semi,
                    )

            fire_idx(0, 0)

            def chunk_body(j, _):
                slot = lax.rem(j, 2)

                @pl.when(j + 1 < nch)
                def _():
                    fire_idx(j + 1, 1 - slot)

                for r_i, role in enumerate(roles):
                    pltpu.make_async_copy(
                        idx_t.at[role, pl.ds(0, 1), pl.ds(j * CH, CH)],
                        idx_c.at[r_i, slot],
                        semi,
                    ).wait()

                for r_i, role in enumerate(roles):
                    @pl.when(j >= 2)
                    def _():
                        pltpu.make_async_copy(
                            out_c.at[r_i, slot],
                            v_hbm.at[role, pl.ds(col, 1), pl.ds((j - 2) * CH, CH)],
                            semo,
                        ).wait()

                    for q in range(qp):
                        iv = idx_c[r_i, slot, 0, pl.ds(q * L, L)]
                        out_c[r_i, slot, 0, pl.ds(q * L, L)] = plsc.load_gather(
                            col_v.at[0], [iv]
                        )
                    pltpu.async_copy(
                        out_c.at[r_i, slot],
                        v_hbm.at[role, pl.ds(col, 1), pl.ds(j * CH, CH)],
                        semo,
                    )
                return 0

            lax.fori_loop(0, nch, chunk_body, 0)

            for j in (nch - 2, nch - 1):
                slot = j % 2
                for r_i, role in enumerate(roles):
                    pltpu.make_async_copy(
                        out_c.at[r_i, slot],
                        v_hbm.at[role, pl.ds(col, 1), pl.ds(j * CH, CH)],
                        semo,
                    ).wait()

        def unit_body(i, _):
            u = wid * UNITS + i
            is_ent = u < E_DIM

            @pl.when(is_ent)
            def _():
                run_unit(ent_t, u, (0, 2))

            @pl.when(jnp.logical_not(is_ent))
            def _():
                run_unit(rel_t, u - E_DIM, (1,))

            return 0

        lax.fori_loop(0, UNITS, unit_body, 0)

    return gcall


def _make_score_call(batch):
    blk = 2048

    def body(vref, oref):
        s = vref[0]
        p = vref[1]
        o = vref[2]
        eps = jnp.float32(1e-12)
        dot = jnp.sum(s * p * o, axis=0, keepdims=True)
        ns = jnp.maximum(jnp.sum(s * s, axis=0, keepdims=True), eps)
        np_ = jnp.maximum(jnp.sum(p * p, axis=0, keepdims=True), eps)
        no = jnp.maximum(jnp.sum(o * o, axis=0, keepdims=True), eps)
        oref[...] = dot * lax.rsqrt(ns) * lax.rsqrt(np_) * lax.rsqrt(no)

    return pl.pallas_call(
        body,
        grid=(batch // blk,),
        in_specs=[pl.BlockSpec((3, E_DIM, blk), lambda i: (0, 0, i))],
        out_specs=pl.BlockSpec((1, blk), lambda i: (0, i)),
        out_shape=jax.ShapeDtypeStruct((1, batch), jnp.float32),
    )


@jax.jit
def kernel(inputs, entity_table, rel_table):
    batch = inputs.shape[0]
    idx_t = jnp.transpose(inputs).reshape(3, 1, batch)
    ent_t = jnp.transpose(entity_table)
    rel_t = jnp.transpose(rel_table)
    v = _make_gather_call(entity_table.shape[0], batch)(ent_t, rel_t, idx_t)
    scores = _make_score_call(batch)(v)
    return jnp.transpose(scores)


# speedup vs baseline: 2.0451x; 1.0302x over previous
import functools

import jax
import jax.numpy as jnp
from jax import lax
from jax.experimental import pallas as pl
from jax.experimental.pallas import tpu as pltpu
from jax.experimental.pallas import tpu_sc as plsc

E_DIM = 64
NC = 2
NS = 16
L = 16
NW = NC * NS
CH = 2048
UNITS = 4


def _make_gather_call(n_rows, batch):
    nch = batch // CH
    qp = CH // L
    mesh = plsc.VectorSubcoreMesh(
        core_axis_name="c", subcore_axis_name="s", num_cores=NC, num_subcores=NS
    )

    @functools.partial(
        pl.kernel,
        out_type=jax.ShapeDtypeStruct((3, E_DIM, batch), jnp.float32),
        mesh=mesh,
        compiler_params=pltpu.CompilerParams(
            needs_layout_passes=False, use_tc_tiling_on_sc=True
        ),
        scratch_types=[
            pltpu.VMEM((1, n_rows), jnp.float32),
            pltpu.VMEM((2, 2, 1, CH), jnp.int32),
            pltpu.VMEM((2, 2, 1, CH), jnp.float32),
            pltpu.SemaphoreType.DMA,
            pltpu.SemaphoreType.DMA,
            pltpu.SemaphoreType.DMA,
        ],
    )
    def gcall(ent_t, rel_t, idx_t, v_hbm, col_v, idx_c, out_c, semc, semi, semo):
        wid = lax.axis_index("s") * NC + lax.axis_index("c")

        def run_unit(tab, col, roles):
            col_cp = pltpu.async_copy(tab.at[pl.ds(col, 1), :], col_v, semc)

            def fire_idx(j, slot):
                for r_i, role in enumerate(roles):
                    pltpu.async_copy(
                        idx_t.at[role, pl.ds(0, 1), pl.ds(j * CH, CH)],
                        idx_c.at[r_i, slot],
                        semi,
                    )

            fire_idx(0, 0)
            col_cp.wait()

            def chunk_body(j, _):
                slot = lax.rem(j, 2)

                @pl.when(j + 1 < nch)
                def _():
                    fire_idx(j + 1, 1 - slot)

                for r_i, role in enumerate(roles):
                    pltpu.make_async_copy(
                        idx_t.at[role, pl.ds(0, 1), pl.ds(j * CH, CH)],
                        idx_c.at[r_i, slot],
                        semi,
                    ).wait()

                for r_i, role in enumerate(roles):
                    @pl.when(j >= 2)
                    def _():
                        pltpu.make_async_copy(
                            out_c.at[r_i, slot],
                            v_hbm.at[role, pl.ds(col, 1), pl.ds((j - 2) * CH, CH)],
                            semo,
                        ).wait()

                    for q in range(qp):
                        iv = idx_c[r_i, slot, 0, pl.ds(q * L, L)]
                        out_c[r_i, slot, 0, pl.ds(q * L, L)] = plsc.load_gather(
                            col_v.at[0], [iv]
                        )
                    pltpu.async_copy(
                        out_c.at[r_i, slot],
                        v_hbm.at[role, pl.ds(col, 1), pl.ds(j * CH, CH)],
                        semo,
                    )
                return 0

            lax.fori_loop(0, nch, chunk_body, 0)

            for j in (nch - 2, nch - 1):
                slot = j % 2
                for r_i, role in enumerate(roles):
                    pltpu.make_async_copy(
                        out_c.at[r_i, slot],
                        v_hbm.at[role, pl.ds(col, 1), pl.ds(j * CH, CH)],
                        semo,
                    ).wait()

        def unit_body(i, _):
            u = wid * UNITS + i
            is_ent = u < E_DIM

            @pl.when(is_ent)
            def _():
                run_unit(ent_t, u, (0, 2))

            @pl.when(jnp.logical_not(is_ent))
            def _():
                run_unit(rel_t, u - E_DIM, (1,))

            return 0

        lax.fori_loop(0, UNITS, unit_body, 0)

    return gcall


def _make_score_call(batch):
    blk = 2048

    def body(vref, oref):
        s = vref[0]
        p = vref[1]
        o = vref[2]
        eps = jnp.float32(1e-12)
        dot = jnp.sum(s * p * o, axis=0, keepdims=True)
        ns = jnp.maximum(jnp.sum(s * s, axis=0, keepdims=True), eps)
        np_ = jnp.maximum(jnp.sum(p * p, axis=0, keepdims=True), eps)
        no = jnp.maximum(jnp.sum(o * o, axis=0, keepdims=True), eps)
        oref[...] = dot * lax.rsqrt(ns) * lax.rsqrt(np_) * lax.rsqrt(no)

    return pl.pallas_call(
        body,
        grid=(batch // blk,),
        in_specs=[pl.BlockSpec((3, E_DIM, blk), lambda i: (0, 0, i))],
        out_specs=pl.BlockSpec((1, blk), lambda i: (0, i)),
        out_shape=jax.ShapeDtypeStruct((1, batch), jnp.float32),
    )


@jax.jit
def kernel(inputs, entity_table, rel_table):
    batch = inputs.shape[0]
    idx_t = jnp.transpose(inputs).reshape(3, 1, batch)
    ent_t = jnp.transpose(entity_table)
    rel_t = jnp.transpose(rel_table)
    v = _make_gather_call(entity_table.shape[0], batch)(ent_t, rel_t, idx_t)
    scores = _make_score_call(batch)(v)
    return jnp.transpose(scores)
